# Initial kernel scaffold; baseline (speedup 1.0000x reference)
import jax
import jax.numpy as jnp
from jax.experimental import pallas as pl


def kernel(z, Wg, bg, W1, b1, W2, b2):
    raise NotImplementedError("write your pallas kernel here")


# speedup vs baseline: 1.2445x; 1.2445x over previous
import functools

import jax
import jax.numpy as jnp
from jax import lax
from jax.experimental import pallas as pl
from jax.experimental.pallas import tpu as pltpu
from jax.experimental.pallas import tpu_sc as plsc

E = 64
K = 2
D = 768
H = 768
T = 2048
BLK = 128
NBLK = 96
HD = D // 2
GW = 128


def _gate_body(z_ref, wg_ref, bg_ref, topw_ref, idx_ref, rank_ref, cnt_ref):
    z = z_ref[...]
    logits = lax.dot_general(z, wg_ref[...], (((1,), (1,)), ((), ())),
                             preferred_element_type=jnp.float32)
    logits = logits + bg_ref[...]
    m = jnp.max(logits, axis=1, keepdims=True)
    ex = jnp.exp(logits - m)
    w = ex / jnp.sum(ex, axis=1, keepdims=True)
    iota = lax.broadcasted_iota(jnp.int32, w.shape, 1)
    m1 = jnp.max(w, axis=1, keepdims=True)
    a1 = jnp.min(jnp.where(w == m1, iota, E), axis=1, keepdims=True)
    wm = jnp.where(iota == a1, -jnp.inf, w)
    m2 = jnp.max(wm, axis=1, keepdims=True)
    a2 = jnp.min(jnp.where(wm == m2, iota, E), axis=1, keepdims=True)
    topw_ref[...] = jnp.concatenate([m1, m2], axis=1)
    idx_ref[...] = jnp.concatenate([a1, a2], axis=1)
    oh1 = (iota == a1).astype(jnp.float32)
    oh2 = (iota == a2).astype(jnp.float32)
    inc = oh1 + oh2
    r = lax.broadcasted_iota(jnp.int32, (T, T), 0)
    c = lax.broadcasted_iota(jnp.int32, (T, T), 1)
    tri = (r > c).astype(jnp.float32)
    excl = lax.dot_general(tri, inc, (((1,), (0,)), ((), ())),
                           preferred_element_type=jnp.float32)
    q1 = jnp.sum(excl * oh1, axis=1, keepdims=True)
    q2 = jnp.sum(excl * oh2, axis=1, keepdims=True)
    rank_ref[...] = jnp.concatenate([q1, q2], axis=1).astype(jnp.int32)
    cnt_ref[...] = jnp.sum(inc, axis=0, keepdims=True).astype(jnp.int32)


def _gate(z, Wg, bg):
    return pl.pallas_call(
        _gate_body,
        grid=(1,),
        in_specs=[
            pl.BlockSpec((T, D), lambda i: (0, 0)),
            pl.BlockSpec((E, D), lambda i: (0, 0)),
            pl.BlockSpec((1, E), lambda i: (0, 0)),
        ],
        out_specs=[
            pl.BlockSpec((T, K), lambda i: (0, 0)),
            pl.BlockSpec((T, K), lambda i: (0, 0)),
            pl.BlockSpec((T, K), lambda i: (0, 0)),
            pl.BlockSpec((1, E), lambda i: (0, 0)),
        ],
        out_shape=[
            jax.ShapeDtypeStruct((T, K), jnp.float32),
            jax.ShapeDtypeStruct((T, K), jnp.int32),
            jax.ShapeDtypeStruct((T, K), jnp.int32),
            jax.ShapeDtypeStruct((1, E), jnp.int32),
        ],
    )(z, Wg, bg.reshape(1, E))


def _sc_gather(z2, gidx2):
    mesh = plsc.VectorSubcoreMesh(core_axis_name="c", subcore_axis_name="s")
    n = NBLK * BLK * 2

    @functools.partial(
        pl.kernel,
        out_type=jax.ShapeDtypeStruct((n, HD), jnp.float32),
        mesh=mesh,
    )
    def k(z_hbm, g_hbm, x_hbm):
        def body(g_vmem, x_vmem):
            pltpu.sync_copy(z_hbm.at[g_vmem.at[0]], x_vmem)

        pltpu.emit_pipeline(
            body,
            grid=(n // GW,),
            in_specs=[pl.BlockSpec((1, GW), lambda i: (0, i))],
            out_specs=[pl.BlockSpec((GW, HD), lambda i: (i, 0))],
            core_axis_name=("c", "s"),
            dimension_semantics=(pltpu.PARALLEL,),
        )(g_hbm, x_hbm)

    return k(z2, gidx2.reshape(1, n))


def _ffn_body(eob_ref, tot_ref, x_ref, w1_ref, b1_ref, w2_ref, b2_ref,
              wp_ref, y_ref):
    j = pl.program_id(0)

    @pl.when(j < tot_ref[0])
    def _():
        x = x_ref[...]
        h = jnp.maximum(
            lax.dot_general(x, w1_ref[0], (((1,), (0,)), ((), ())),
                            preferred_element_type=jnp.float32)
            + b1_ref[0], 0.0)
        y = lax.dot_general(h, w2_ref[0], (((1,), (0,)), ((), ())),
                            preferred_element_type=jnp.float32) + b2_ref[0]
        y_ref[...] = y * wp_ref[0]


def _ffn(X, W1, b1, W2, b2, Wpad, eob, tot):
    grid_spec = pltpu.PrefetchScalarGridSpec(
        num_scalar_prefetch=2,
        grid=(NBLK,),
        in_specs=[
            pl.BlockSpec((BLK, D), lambda j, eob, tot: (j, 0)),
            pl.BlockSpec((1, D, H), lambda j, eob, tot: (eob[j], 0, 0)),
            pl.BlockSpec((1, 1, H), lambda j, eob, tot: (eob[j], 0, 0)),
            pl.BlockSpec((1, H, D), lambda j, eob, tot: (eob[j], 0, 0)),
            pl.BlockSpec((1, 1, D), lambda j, eob, tot: (eob[j], 0, 0)),
            pl.BlockSpec((1, BLK, 1), lambda j, eob, tot: (j, 0, 0)),
        ],
        out_specs=pl.BlockSpec((BLK, D), lambda j, eob, tot: (j, 0)),
    )
    return pl.pallas_call(
        _ffn_body,
        grid_spec=grid_spec,
        out_shape=jax.ShapeDtypeStruct((NBLK * BLK, D), jnp.float32),
    )(eob, tot, X, W1, b1.reshape(E, 1, H), W2, b2.reshape(E, 1, D),
      Wpad.reshape(NBLK, BLK, 1))


def _sc_combine(Y2, p0h, p1h):
    mesh = plsc.VectorSubcoreMesh(core_axis_name="c", subcore_axis_name="s")
    nw = 32
    per = T * 2 // nw

    @functools.partial(
        pl.kernel,
        out_type=jax.ShapeDtypeStruct((T * 2, HD), jnp.float32),
        mesh=mesh,
        scratch_types=[
            pltpu.VMEM((per,), jnp.int32),
            pltpu.VMEM((per,), jnp.int32),
            pltpu.VMEM((per, HD), jnp.float32),
            pltpu.VMEM((per, HD), jnp.float32),
            pltpu.SemaphoreType.DMA,
        ],
    )
    def k(y_hbm, p0_hbm, p1_hbm, o_hbm, i0, i1, b0, b1, sem):
        wid = lax.axis_index("s") * 2 + lax.axis_index("c")
        base = wid * per
        pltpu.sync_copy(p0_hbm.at[pl.ds(base, per)], i0)
        pltpu.sync_copy(p1_hbm.at[pl.ds(base, per)], i1)
        pltpu.async_copy(y_hbm.at[i0], b0, sem).wait()
        pltpu.async_copy(y_hbm.at[i1], b1, sem).wait()

        @pl.loop(0, per)
        def _(i):
            @pl.loop(0, HD, step=16)
            def _(c):
                b0[i, pl.ds(c, 16)] = b0[i, pl.ds(c, 16)] + b1[i, pl.ds(c, 16)]

        pltpu.sync_copy(b0, o_hbm.at[pl.ds(base, per)])

    return k(Y2, p0h, p1h)


def kernel(z, Wg, bg, W1, b1, W2, b2):
    topw, idx, rank, cnt = _gate(z, Wg, bg)

    counts = cnt[0]
    nb = (counts + (BLK - 1)) // BLK
    cnb = jnp.cumsum(nb)
    total = cnb[E - 1]
    bs = cnb - nb
    eobs = jnp.searchsorted(cnb, jnp.arange(NBLK), side="right")
    last_e = jnp.take(eobs, total - 1)
    active = jnp.arange(NBLK) < total
    eob = jnp.where(active, eobs, last_e).astype(jnp.int32)

    e_flat = idx.reshape(-1)
    q_flat = rank.reshape(-1)
    w_flat = topw.reshape(-1)
    t_flat = jnp.repeat(jnp.arange(T, dtype=jnp.int32), K)
    pp = jnp.take(bs, e_flat).astype(jnp.int32) * BLK + q_flat
    gidx = jnp.zeros((NBLK * BLK,), jnp.int32).at[pp].set(t_flat)
    wpad = jnp.zeros((NBLK * BLK,), jnp.float32).at[pp].set(w_flat)
    pos = pp.reshape(T, K)

    two = jnp.arange(K, dtype=jnp.int32)
    gidx2 = (2 * gidx[:, None] + two).reshape(-1)
    p0h = (2 * pos[:, 0:1] + two).reshape(-1)
    p1h = (2 * pos[:, 1:2] + two).reshape(-1)

    X2 = _sc_gather(z.reshape(T * 2, HD), gidx2)
    Y = _ffn(X2.reshape(NBLK * BLK, D), W1, b1, W2, b2, wpad, eob,
             total.reshape(1).astype(jnp.int32))
    out2 = _sc_combine(Y.reshape(NBLK * BLK * 2, HD), p0h, p1h)
    return out2.reshape(T, D)


# speedup vs baseline: 1.3534x; 1.0876x over previous
import functools

import jax
import jax.numpy as jnp
from jax import lax
from jax.experimental import pallas as pl
from jax.experimental.pallas import tpu as pltpu
from jax.experimental.pallas import tpu_sc as plsc

E = 64
K = 2
D = 768
H = 768
T = 2048
BLK = 128
NBLK = 96
HD = D // 2
GW = 128


def _gate_body(z_ref, wg_ref, bg_ref, topw_ref, idx_ref, rank_ref, cnt_ref):
    z = z_ref[...]
    logits = lax.dot_general(z, wg_ref[...], (((1,), (1,)), ((), ())),
                             preferred_element_type=jnp.float32)
    logits = logits + bg_ref[...]
    m = jnp.max(logits, axis=1, keepdims=True)
    ex = jnp.exp(logits - m)
    w = ex / jnp.sum(ex, axis=1, keepdims=True)
    iota = lax.broadcasted_iota(jnp.int32, w.shape, 1)
    m1 = jnp.max(w, axis=1, keepdims=True)
    a1 = jnp.min(jnp.where(w == m1, iota, E), axis=1, keepdims=True)
    wm = jnp.where(iota == a1, -jnp.inf, w)
    m2 = jnp.max(wm, axis=1, keepdims=True)
    a2 = jnp.min(jnp.where(wm == m2, iota, E), axis=1, keepdims=True)
    topw_ref[...] = jnp.concatenate([m1, m2], axis=1)
    idx_ref[...] = jnp.concatenate([a1, a2], axis=1)
    oh1 = (iota == a1).astype(jnp.float32)
    oh2 = (iota == a2).astype(jnp.float32)
    inc = oh1 + oh2
    r = lax.broadcasted_iota(jnp.int32, (T, T), 0)
    c = lax.broadcasted_iota(jnp.int32, (T, T), 1)
    tri = (r > c).astype(jnp.float32)
    excl = lax.dot_general(tri, inc, (((1,), (0,)), ((), ())),
                           preferred_element_type=jnp.float32)
    q1 = jnp.sum(excl * oh1, axis=1, keepdims=True)
    q2 = jnp.sum(excl * oh2, axis=1, keepdims=True)
    rank_ref[...] = jnp.concatenate([q1, q2], axis=1).astype(jnp.int32)
    cnt_ref[...] = jnp.sum(inc, axis=0, keepdims=True).astype(jnp.int32)


def _gate(z, Wg, bg):
    return pl.pallas_call(
        _gate_body,
        grid=(1,),
        in_specs=[
            pl.BlockSpec((T, D), lambda i: (0, 0)),
            pl.BlockSpec((E, D), lambda i: (0, 0)),
            pl.BlockSpec((1, E), lambda i: (0, 0)),
        ],
        out_specs=[
            pl.BlockSpec((T, K), lambda i: (0, 0)),
            pl.BlockSpec((T, K), lambda i: (0, 0)),
            pl.BlockSpec((T, K), lambda i: (0, 0)),
            pl.BlockSpec((1, E), lambda i: (0, 0)),
        ],
        out_shape=[
            jax.ShapeDtypeStruct((T, K), jnp.float32),
            jax.ShapeDtypeStruct((T, K), jnp.int32),
            jax.ShapeDtypeStruct((T, K), jnp.int32),
            jax.ShapeDtypeStruct((1, E), jnp.int32),
        ],
    )(z, Wg, bg.reshape(1, E))


def _sc_gather(z2, gidx2):
    mesh = plsc.VectorSubcoreMesh(core_axis_name="c", subcore_axis_name="s")
    n = NBLK * BLK * 2
    nw = 32
    per = n // nw
    nch = per // GW

    @functools.partial(
        pl.kernel,
        out_type=jax.ShapeDtypeStruct((n, HD), jnp.float32),
        mesh=mesh,
        scratch_types=[
            pltpu.VMEM((8, GW), jnp.int32),
            pltpu.VMEM((2, GW, HD), jnp.float32),
            pltpu.SemaphoreType.DMA,
            pltpu.SemaphoreType.DMA,
        ],
    )
    def k(z_hbm, g_hbm, x_hbm, idxs, bufs, sem0, sem1):
        wid = lax.axis_index("s") * 2 + lax.axis_index("c")
        pltpu.sync_copy(g_hbm.at[wid], idxs)
        sems = (sem0, sem1)
        cps = [None, None]
        cps[0] = pltpu.async_copy(z_hbm.at[idxs.at[0]], bufs.at[0], sem0)
        for c in range(nch):
            cur = c % 2
            nxt = (c + 1) % 2
            if c + 1 < nch:
                cps[nxt] = pltpu.async_copy(
                    z_hbm.at[idxs.at[c + 1]], bufs.at[nxt], sems[nxt])
            cps[cur].wait()
            pltpu.sync_copy(bufs.at[cur],
                            x_hbm.at[pl.ds(wid * per + c * GW, GW)])

    gp = jnp.zeros((nw, 8, GW), jnp.int32)
    gp = gp.at[:, :nch, :].set(gidx2.reshape(nw, nch, GW))
    return k(z2, gp)


def _ffn_body(eob_ref, tot_ref, x_ref, w1_ref, b1_ref, w2_ref, b2_ref,
              wp_ref, y_ref):
    j = pl.program_id(0)

    @pl.when(j < tot_ref[0])
    def _():
        x = x_ref[...]
        h = jnp.maximum(
            lax.dot_general(x, w1_ref[0], (((1,), (0,)), ((), ())),
                            preferred_element_type=jnp.float32)
            + b1_ref[0], 0.0)
        y = lax.dot_general(h, w2_ref[0], (((1,), (0,)), ((), ())),
                            preferred_element_type=jnp.float32) + b2_ref[0]
        y_ref[...] = y * wp_ref[0]


def _ffn(X, W1, b1, W2, b2, Wpad, eob, tot):
    grid_spec = pltpu.PrefetchScalarGridSpec(
        num_scalar_prefetch=2,
        grid=(NBLK,),
        in_specs=[
            pl.BlockSpec((BLK, D), lambda j, eob, tot: (j, 0)),
            pl.BlockSpec((1, D, H), lambda j, eob, tot: (eob[j], 0, 0)),
            pl.BlockSpec((1, 1, H), lambda j, eob, tot: (eob[j], 0, 0)),
            pl.BlockSpec((1, H, D), lambda j, eob, tot: (eob[j], 0, 0)),
            pl.BlockSpec((1, 1, D), lambda j, eob, tot: (eob[j], 0, 0)),
            pl.BlockSpec((1, BLK, 1), lambda j, eob, tot: (j, 0, 0)),
        ],
        out_specs=pl.BlockSpec((BLK, D), lambda j, eob, tot: (j, 0)),
    )
    return pl.pallas_call(
        _ffn_body,
        grid_spec=grid_spec,
        out_shape=jax.ShapeDtypeStruct((NBLK * BLK, D), jnp.float32),
    )(eob, tot, X, W1, b1.reshape(E, 1, H), W2, b2.reshape(E, 1, D),
      Wpad.reshape(NBLK, BLK, 1))


def _sc_combine(Y2, p0h, p1h):
    mesh = plsc.VectorSubcoreMesh(core_axis_name="c", subcore_axis_name="s")
    nw = 32
    per = T * 2 // nw

    @functools.partial(
        pl.kernel,
        out_type=jax.ShapeDtypeStruct((T * 2, HD), jnp.float32),
        mesh=mesh,
        scratch_types=[
            pltpu.VMEM((per,), jnp.int32),
            pltpu.VMEM((per,), jnp.int32),
            pltpu.VMEM((per, HD), jnp.float32),
            pltpu.VMEM((per, HD), jnp.float32),
            pltpu.SemaphoreType.DMA,
        ],
    )
    def k(y_hbm, p0_hbm, p1_hbm, o_hbm, i0, i1, b0, b1, sem):
        wid = lax.axis_index("s") * 2 + lax.axis_index("c")
        base = wid * per
        pltpu.sync_copy(p0_hbm.at[pl.ds(base, per)], i0)
        pltpu.sync_copy(p1_hbm.at[pl.ds(base, per)], i1)
        pltpu.async_copy(y_hbm.at[i0], b0, sem).wait()
        pltpu.async_copy(y_hbm.at[i1], b1, sem).wait()

        @pl.loop(0, per)
        def _(i):
            @pl.loop(0, HD, step=16)
            def _(c):
                b0[i, pl.ds(c, 16)] = b0[i, pl.ds(c, 16)] + b1[i, pl.ds(c, 16)]

        pltpu.sync_copy(b0, o_hbm.at[pl.ds(base, per)])

    return k(Y2, p0h, p1h)


def kernel(z, Wg, bg, W1, b1, W2, b2):
    topw, idx, rank, cnt = _gate(z, Wg, bg)

    counts = cnt[0]
    nb = (counts + (BLK - 1)) // BLK
    cnb = jnp.cumsum(nb)
    total = cnb[E - 1]
    bs = cnb - nb
    eobs = jnp.searchsorted(cnb, jnp.arange(NBLK), side="right")
    last_e = jnp.take(eobs, total - 1)
    active = jnp.arange(NBLK) < total
    eob = jnp.where(active, eobs, last_e).astype(jnp.int32)

    e_flat = idx.reshape(-1)
    q_flat = rank.reshape(-1)
    w_flat = topw.reshape(-1)
    t_flat = jnp.repeat(jnp.arange(T, dtype=jnp.int32), K)
    pp = jnp.take(bs, e_flat).astype(jnp.int32) * BLK + q_flat
    gidx = jnp.zeros((NBLK * BLK,), jnp.int32).at[pp].set(t_flat)
    wpad = jnp.zeros((NBLK * BLK,), jnp.float32).at[pp].set(w_flat)
    pos = pp.reshape(T, K)

    two = jnp.arange(K, dtype=jnp.int32)
    gidx2 = (2 * gidx[:, None] + two).reshape(-1)
    p0h = (2 * pos[:, 0:1] + two).reshape(-1)
    p1h = (2 * pos[:, 1:2] + two).reshape(-1)

    X2 = _sc_gather(z.reshape(T * 2, HD), gidx2)
    Y = _ffn(X2.reshape(NBLK * BLK, D), W1, b1, W2, b2, wpad, eob,
             total.reshape(1).astype(jnp.int32))
    out2 = _sc_combine(Y.reshape(NBLK * BLK * 2, HD), p0h, p1h)
    return out2.reshape(T, D)


# speedup vs baseline: 2.5399x; 1.8766x over previous
import functools

import jax
import jax.numpy as jnp
from jax import lax
from jax.experimental import pallas as pl
from jax.experimental.pallas import tpu as pltpu
from jax.experimental.pallas import tpu_sc as plsc

E = 64
K = 2
D = 768
H = 768
T = 2048
BLK = 128
NBLK = 96
HD = D // 2
GW = 128


def _gate_body(z_ref, wg_ref, bg_ref, topw_ref, idx_ref, rank_ref, cnt_ref):
    z = z_ref[...]
    logits = lax.dot_general(z, wg_ref[...], (((1,), (1,)), ((), ())),
                             preferred_element_type=jnp.float32)
    logits = logits + bg_ref[...]
    m = jnp.max(logits, axis=1, keepdims=True)
    ex = jnp.exp(logits - m)
    w = ex / jnp.sum(ex, axis=1, keepdims=True)
    iota = lax.broadcasted_iota(jnp.int32, w.shape, 1)
    m1 = jnp.max(w, axis=1, keepdims=True)
    a1 = jnp.min(jnp.where(w == m1, iota, E), axis=1, keepdims=True)
    wm = jnp.where(iota == a1, -jnp.inf, w)
    m2 = jnp.max(wm, axis=1, keepdims=True)
    a2 = jnp.min(jnp.where(wm == m2, iota, E), axis=1, keepdims=True)
    topw_ref[...] = jnp.concatenate([m1, m2], axis=1)
    idx_ref[...] = jnp.concatenate([a1, a2], axis=1)
    oh1 = (iota == a1).astype(jnp.float32)
    oh2 = (iota == a2).astype(jnp.float32)
    inc = oh1 + oh2
    r = lax.broadcasted_iota(jnp.int32, (T, T), 0)
    c = lax.broadcasted_iota(jnp.int32, (T, T), 1)
    tri = (r > c).astype(jnp.float32)
    excl = lax.dot_general(tri, inc, (((1,), (0,)), ((), ())),
                           preferred_element_type=jnp.float32)
    q1 = jnp.sum(excl * oh1, axis=1, keepdims=True)
    q2 = jnp.sum(excl * oh2, axis=1, keepdims=True)
    rank_ref[...] = jnp.concatenate([q1, q2], axis=1).astype(jnp.int32)
    cnt_ref[...] = jnp.sum(inc, axis=0, keepdims=True).astype(jnp.int32)


def _gate(z, Wg, bg):
    return pl.pallas_call(
        _gate_body,
        grid=(1,),
        in_specs=[
            pl.BlockSpec((T, D), lambda i: (0, 0)),
            pl.BlockSpec((E, D), lambda i: (0, 0)),
            pl.BlockSpec((1, E), lambda i: (0, 0)),
        ],
        out_specs=[
            pl.BlockSpec((T, K), lambda i: (0, 0)),
            pl.BlockSpec((T, K), lambda i: (0, 0)),
            pl.BlockSpec((T, K), lambda i: (0, 0)),
            pl.BlockSpec((1, E), lambda i: (0, 0)),
        ],
        out_shape=[
            jax.ShapeDtypeStruct((T, K), jnp.float32),
            jax.ShapeDtypeStruct((T, K), jnp.int32),
            jax.ShapeDtypeStruct((T, K), jnp.int32),
            jax.ShapeDtypeStruct((1, E), jnp.int32),
        ],
    )(z, Wg, bg.reshape(1, E))


def _sc_gather(z2, gidx2):
    mesh = plsc.VectorSubcoreMesh(core_axis_name="c", subcore_axis_name="s")
    n = NBLK * BLK * 2
    nw = 32
    per = n // nw
    nch = per // GW

    @functools.partial(
        pl.kernel,
        out_type=jax.ShapeDtypeStruct((n, HD), jnp.float32),
        mesh=mesh,
        scratch_types=[
            pltpu.VMEM((8, GW), jnp.int32),
            pltpu.VMEM((2, GW, HD), jnp.float32),
            pltpu.SemaphoreType.DMA,
            pltpu.SemaphoreType.DMA,
        ],
    )
    def k(z_hbm, g_hbm, x_hbm, idxs, bufs, sem0, sem1):
        wid = lax.axis_index("s") * 2 + lax.axis_index("c")
        pltpu.sync_copy(g_hbm.at[wid], idxs)
        sems = (sem0, sem1)
        cps = [None, None]
        cps[0] = pltpu.async_copy(z_hbm.at[idxs.at[0]], bufs.at[0], sem0)
        for c in range(nch):
            cur = c % 2
            nxt = (c + 1) % 2
            if c + 1 < nch:
                cps[nxt] = pltpu.async_copy(
                    z_hbm.at[idxs.at[c + 1]], bufs.at[nxt], sems[nxt])
            cps[cur].wait()
            pltpu.sync_copy(bufs.at[cur],
                            x_hbm.at[pl.ds(wid * per + c * GW, GW)])

    gp = jnp.zeros((nw, 8, GW), jnp.int32)
    gp = gp.at[:, :nch, :].set(gidx2.reshape(nw, nch, GW))
    return k(z2, gp)


def _ffn_body(eob_ref, tot_ref, x_ref, w1_ref, b1_ref, w2_ref, b2_ref,
              wp_ref, y_ref):
    j = pl.program_id(0)

    @pl.when(j < tot_ref[0])
    def _():
        x = x_ref[...]
        h = jnp.maximum(
            lax.dot_general(x, w1_ref[0], (((1,), (0,)), ((), ())),
                            preferred_element_type=jnp.float32)
            + b1_ref[0], 0.0)
        y = lax.dot_general(h, w2_ref[0], (((1,), (0,)), ((), ())),
                            preferred_element_type=jnp.float32) + b2_ref[0]
        y_ref[...] = y * wp_ref[0]


def _ffn(X, W1, b1, W2, b2, Wpad, eob, tot):
    grid_spec = pltpu.PrefetchScalarGridSpec(
        num_scalar_prefetch=2,
        grid=(NBLK,),
        in_specs=[
            pl.BlockSpec((BLK, D), lambda j, eob, tot: (j, 0)),
            pl.BlockSpec((1, D, H), lambda j, eob, tot: (eob[j], 0, 0)),
            pl.BlockSpec((1, 1, H), lambda j, eob, tot: (eob[j], 0, 0)),
            pl.BlockSpec((1, H, D), lambda j, eob, tot: (eob[j], 0, 0)),
            pl.BlockSpec((1, 1, D), lambda j, eob, tot: (eob[j], 0, 0)),
            pl.BlockSpec((1, BLK, 1), lambda j, eob, tot: (j, 0, 0)),
        ],
        out_specs=pl.BlockSpec((BLK, D), lambda j, eob, tot: (j, 0)),
    )
    return pl.pallas_call(
        _ffn_body,
        grid_spec=grid_spec,
        out_shape=jax.ShapeDtypeStruct((NBLK * BLK, D), jnp.float32),
    )(eob, tot, X, W1, b1.reshape(E, 1, H), W2, b2.reshape(E, 1, D),
      Wpad.reshape(NBLK, BLK, 1))


def _sc_combine(Y2, p0h, p1h):
    mesh = plsc.VectorSubcoreMesh(core_axis_name="c", subcore_axis_name="s")
    nw = 32
    per = T * 2 // nw

    @functools.partial(
        pl.kernel,
        out_type=jax.ShapeDtypeStruct((T * 2, HD), jnp.float32),
        mesh=mesh,
        scratch_types=[
            pltpu.VMEM((per,), jnp.int32),
            pltpu.VMEM((per,), jnp.int32),
            pltpu.VMEM((per, HD), jnp.float32),
            pltpu.VMEM((per, HD), jnp.float32),
            pltpu.SemaphoreType.DMA,
        ],
    )
    def k(y_hbm, p0_hbm, p1_hbm, o_hbm, i0, i1, b0, b1, sem):
        wid = lax.axis_index("s") * 2 + lax.axis_index("c")
        base = wid * per
        pltpu.sync_copy(p0_hbm.at[pl.ds(base, per)], i0)
        pltpu.sync_copy(p1_hbm.at[pl.ds(base, per)], i1)
        pltpu.async_copy(y_hbm.at[i0], b0, sem).wait()
        pltpu.async_copy(y_hbm.at[i1], b1, sem).wait()

        @pl.loop(0, per)
        def _(i):
            @pl.loop(0, HD, step=16)
            def _(c):
                b0[i, pl.ds(c, 16)] = b0[i, pl.ds(c, 16)] + b1[i, pl.ds(c, 16)]

        pltpu.sync_copy(b0, o_hbm.at[pl.ds(base, per)])

    return k(Y2, p0h, p1h)


def kernel(z, Wg, bg, W1, b1, W2, b2):
    topw, idx, rank, cnt = _gate(z, Wg, bg)

    counts = cnt[0]
    nb = (counts + (BLK - 1)) // BLK
    cnb = jnp.cumsum(nb)
    total = cnb[E - 1]
    bs = cnb - nb
    eobs = jnp.searchsorted(cnb, jnp.arange(NBLK), side="right")
    last_e = jnp.take(eobs, total - 1)
    active = jnp.arange(NBLK) < total
    eob = jnp.where(active, eobs, last_e).astype(jnp.int32)

    e_flat = idx.reshape(-1)
    q_flat = rank.reshape(-1)
    w_flat = topw.reshape(-1)
    t_flat = jnp.repeat(jnp.arange(T, dtype=jnp.int32), K)
    pp = jnp.take(bs, e_flat).astype(jnp.int32) * BLK + q_flat
    gidx = (jnp.arange(NBLK * BLK, dtype=jnp.int32) % T).at[pp].set(t_flat)
    wpad = jnp.zeros((NBLK * BLK,), jnp.float32).at[pp].set(w_flat)
    pos = pp.reshape(T, K)

    two = jnp.arange(K, dtype=jnp.int32)
    gidx2 = (2 * gidx[:, None] + two).reshape(-1)
    p0h = (2 * pos[:, 0:1] + two).reshape(-1)
    p1h = (2 * pos[:, 1:2] + two).reshape(-1)

    X2 = _sc_gather(z.reshape(T * 2, HD), gidx2)
    Y = _ffn(X2.reshape(NBLK * BLK, D), W1, b1, W2, b2, wpad, eob,
             total.reshape(1).astype(jnp.int32))
    out2 = _sc_combine(Y.reshape(NBLK * BLK * 2, HD), p0h, p1h)
    return out2.reshape(T, D)


# speedup vs baseline: 3.4001x; 1.3387x over previous
import functools

import jax
import jax.numpy as jnp
from jax import lax
from jax.experimental import pallas as pl
from jax.experimental.pallas import tpu as pltpu
from jax.experimental.pallas import tpu_sc as plsc

E = 64
K = 2
D = 768
H = 768
T = 2048
BLK = 128
NBLK = 96
GW = 128


def _gate_body(z_ref, wg_ref, bg_ref, topw_ref, idx_ref, rank_ref, cnt_ref):
    z = z_ref[...]
    logits = lax.dot_general(z, wg_ref[...], (((1,), (1,)), ((), ())),
                             preferred_element_type=jnp.float32)
    logits = logits + bg_ref[...]
    m = jnp.max(logits, axis=1, keepdims=True)
    ex = jnp.exp(logits - m)
    w = ex / jnp.sum(ex, axis=1, keepdims=True)
    iota = lax.broadcasted_iota(jnp.int32, w.shape, 1)
    m1 = jnp.max(w, axis=1, keepdims=True)
    a1 = jnp.min(jnp.where(w == m1, iota, E), axis=1, keepdims=True)
    wm = jnp.where(iota == a1, -jnp.inf, w)
    m2 = jnp.max(wm, axis=1, keepdims=True)
    a2 = jnp.min(jnp.where(wm == m2, iota, E), axis=1, keepdims=True)
    topw_ref[...] = jnp.concatenate([m1, m2], axis=1)
    idx_ref[...] = jnp.concatenate([a1, a2], axis=1)
    oh1 = (iota == a1).astype(jnp.float32)
    oh2 = (iota == a2).astype(jnp.float32)
    inc = oh1 + oh2
    r = lax.broadcasted_iota(jnp.int32, (T, T), 0)
    c = lax.broadcasted_iota(jnp.int32, (T, T), 1)
    tri = (r > c).astype(jnp.float32)
    excl = lax.dot_general(tri, inc, (((1,), (0,)), ((), ())),
                           preferred_element_type=jnp.float32)
    q1 = jnp.sum(excl * oh1, axis=1, keepdims=True)
    q2 = jnp.sum(excl * oh2, axis=1, keepdims=True)
    rank_ref[...] = jnp.concatenate([q1, q2], axis=1).astype(jnp.int32)
    cnt_ref[...] = jnp.sum(inc, axis=0, keepdims=True).astype(jnp.int32)


def _gate(z, Wg, bg):
    return pl.pallas_call(
        _gate_body,
        grid=(1,),
        in_specs=[
            pl.BlockSpec((T, D), lambda i: (0, 0)),
            pl.BlockSpec((E, D), lambda i: (0, 0)),
            pl.BlockSpec((1, E), lambda i: (0, 0)),
        ],
        out_specs=[
            pl.BlockSpec((T, K), lambda i: (0, 0)),
            pl.BlockSpec((T, K), lambda i: (0, 0)),
            pl.BlockSpec((T, K), lambda i: (0, 0)),
            pl.BlockSpec((1, E), lambda i: (0, 0)),
        ],
        out_shape=[
            jax.ShapeDtypeStruct((T, K), jnp.float32),
            jax.ShapeDtypeStruct((T, K), jnp.int32),
            jax.ShapeDtypeStruct((T, K), jnp.int32),
            jax.ShapeDtypeStruct((1, E), jnp.int32),
        ],
    )(z, Wg, bg.reshape(1, E))


def _sc_gather(z, gidx):
    mesh = plsc.VectorSubcoreMesh(core_axis_name="c", subcore_axis_name="s")
    n = NBLK * BLK
    nw = 32
    per = n // nw
    nch = per // GW

    @functools.partial(
        pl.kernel,
        out_type=jax.ShapeDtypeStruct((n, D), jnp.float32),
        mesh=mesh,
        scratch_types=[
            pltpu.VMEM((8, GW), jnp.int32),
            pltpu.VMEM((GW, D), jnp.float32),
            pltpu.SemaphoreType.DMA,
        ],
    )
    def k(z_hbm, g_hbm, x_hbm, idxs, buf, sem):
        wid = lax.axis_index("s") * 2 + lax.axis_index("c")
        pltpu.sync_copy(g_hbm.at[wid], idxs)
        for c in range(nch):
            pltpu.async_copy(z_hbm.at[idxs.at[c]], buf, sem).wait()
            pltpu.sync_copy(buf, x_hbm.at[pl.ds(wid * per + c * GW, GW)])

    gp = jnp.zeros((nw, 8, GW), jnp.int32)
    gp = gp.at[:, :nch, :].set(gidx.reshape(nw, nch, GW))
    return k(z, gp)


def _ffn_body(eob_ref, tot_ref, x_ref, w1_ref, b1_ref, w2_ref, b2_ref,
              wp_ref, y_ref):
    j = pl.program_id(0)

    @pl.when(j < tot_ref[0])
    def _():
        x = x_ref[...]
        h = jnp.maximum(
            lax.dot_general(x, w1_ref[0], (((1,), (0,)), ((), ())),
                            preferred_element_type=jnp.float32)
            + b1_ref[0], 0.0)
        y = lax.dot_general(h, w2_ref[0], (((1,), (0,)), ((), ())),
                            preferred_element_type=jnp.float32) + b2_ref[0]
        y_ref[...] = y * wp_ref[0]


def _ffn(X, W1, b1, W2, b2, Wpad, eob, tot):
    grid_spec = pltpu.PrefetchScalarGridSpec(
        num_scalar_prefetch=2,
        grid=(NBLK,),
        in_specs=[
            pl.BlockSpec((BLK, D), lambda j, eob, tot: (j, 0)),
            pl.BlockSpec((1, D, H), lambda j, eob, tot: (eob[j], 0, 0)),
            pl.BlockSpec((1, 1, H), lambda j, eob, tot: (eob[j], 0, 0)),
            pl.BlockSpec((1, H, D), lambda j, eob, tot: (eob[j], 0, 0)),
            pl.BlockSpec((1, 1, D), lambda j, eob, tot: (eob[j], 0, 0)),
            pl.BlockSpec((1, BLK, 1), lambda j, eob, tot: (j, 0, 0)),
        ],
        out_specs=pl.BlockSpec((BLK, D), lambda j, eob, tot: (j, 0)),
    )
    return pl.pallas_call(
        _ffn_body,
        grid_spec=grid_spec,
        out_shape=jax.ShapeDtypeStruct((NBLK * BLK, D), jnp.float32),
    )(eob, tot, X, W1, b1.reshape(E, 1, H), W2, b2.reshape(E, 1, D),
      Wpad.reshape(NBLK, BLK, 1))


def _sc_combine(Y, p0, p1):
    mesh = plsc.VectorSubcoreMesh(core_axis_name="c", subcore_axis_name="s")
    nw = 32
    per = T // nw

    @functools.partial(
        pl.kernel,
        out_type=jax.ShapeDtypeStruct((T, D), jnp.float32),
        mesh=mesh,
        scratch_types=[
            pltpu.VMEM((per,), jnp.int32),
            pltpu.VMEM((per,), jnp.int32),
            pltpu.VMEM((per, D), jnp.float32),
            pltpu.VMEM((per, D), jnp.float32),
            pltpu.SemaphoreType.DMA,
        ],
    )
    def k(y_hbm, p0_hbm, p1_hbm, o_hbm, i0, i1, b0, b1, sem):
        wid = lax.axis_index("s") * 2 + lax.axis_index("c")
        base = wid * per
        pltpu.sync_copy(p0_hbm.at[pl.ds(base, per)], i0)
        pltpu.sync_copy(p1_hbm.at[pl.ds(base, per)], i1)
        pltpu.async_copy(y_hbm.at[i0], b0, sem).wait()
        pltpu.async_copy(y_hbm.at[i1], b1, sem).wait()

        @pl.loop(0, per)
        def _(i):
            @pl.loop(0, D, step=16)
            def _(c):
                b0[i, pl.ds(c, 16)] = b0[i, pl.ds(c, 16)] + b1[i, pl.ds(c, 16)]

        pltpu.sync_copy(b0, o_hbm.at[pl.ds(base, per)])

    return k(Y, p0, p1)


def kernel(z, Wg, bg, W1, b1, W2, b2):
    topw, idx, rank, cnt = _gate(z, Wg, bg)

    counts = cnt[0]
    nb = (counts + (BLK - 1)) // BLK
    cnb = jnp.cumsum(nb)
    total = cnb[E - 1]
    bs = cnb - nb
    eobs = jnp.searchsorted(cnb, jnp.arange(NBLK), side="right")
    last_e = jnp.take(eobs, total - 1)
    active = jnp.arange(NBLK) < total
    eob = jnp.where(active, eobs, last_e).astype(jnp.int32)

    e_flat = idx.reshape(-1)
    q_flat = rank.reshape(-1)
    w_flat = topw.reshape(-1)
    t_flat = jnp.repeat(jnp.arange(T, dtype=jnp.int32), K)
    pp = jnp.take(bs, e_flat).astype(jnp.int32) * BLK + q_flat
    gidx = (jnp.arange(NBLK * BLK, dtype=jnp.int32) % T).at[pp].set(t_flat)
    wpad = jnp.zeros((NBLK * BLK,), jnp.float32).at[pp].set(w_flat)
    pos = pp.reshape(T, K)

    X = _sc_gather(z, gidx)
    Y = _ffn(X, W1, b1, W2, b2, wpad, eob, total.reshape(1).astype(jnp.int32))
    return _sc_combine(Y, pos[:, 0], pos[:, 1])


# speedup vs baseline: 4.0835x; 1.2010x over previous
import dataclasses
import functools

import jax
import jax.numpy as jnp
from jax import lax
from jax.experimental import pallas as pl
from jax.experimental.pallas import tpu as pltpu
from jax.experimental.pallas import tpu_sc as plsc

E = 64
K = 2
D = 768
H = 768
T = 2048
BLK = 128
NBLK = 96
GW = 128


def _sc_params():
    cp = pltpu.CompilerParams()
    if "needs_layout_passes" in pltpu.CompilerParams.__dataclass_fields__:
        cp = dataclasses.replace(cp, needs_layout_passes=False)
    return cp


def _gate_body(z_ref, wg_ref, bg_ref, pos0_ref, pos1_ref, w0_ref, w1_ref,
               rr_ref, cc_ref, eob_ref, tot_ref):
    z = z_ref[...]
    logits = lax.dot_general(z, wg_ref[...], (((1,), (1,)), ((), ())),
                             preferred_element_type=jnp.float32)
    logits = logits + bg_ref[...]
    m = jnp.max(logits, axis=1, keepdims=True)
    ex = jnp.exp(logits - m)
    w = ex / jnp.sum(ex, axis=1, keepdims=True)
    iota = lax.broadcasted_iota(jnp.int32, w.shape, 1)
    m1 = jnp.max(w, axis=1, keepdims=True)
    a1 = jnp.min(jnp.where(w == m1, iota, E), axis=1, keepdims=True)
    wm = jnp.where(iota == a1, -jnp.inf, w)
    m2 = jnp.max(wm, axis=1, keepdims=True)
    a2 = jnp.min(jnp.where(wm == m2, iota, E), axis=1, keepdims=True)
    oh1 = (iota == a1).astype(jnp.float32)
    oh2 = (iota == a2).astype(jnp.float32)
    inc = oh1 + oh2
    r = lax.broadcasted_iota(jnp.int32, (T, T), 0)
    c = lax.broadcasted_iota(jnp.int32, (T, T), 1)
    tri = (r > c).astype(jnp.float32)
    eye = (r == c).astype(jnp.float32)
    excl = lax.dot_general(tri, inc, (((1,), (0,)), ((), ())),
                           preferred_element_type=jnp.float32)
    q1 = jnp.floor(jnp.sum(excl * oh1, axis=1, keepdims=True) + 0.5)
    q2 = jnp.floor(jnp.sum(excl * oh2, axis=1, keepdims=True) + 0.5)
    ones_col = jnp.ones((T, 1), jnp.float32)
    counts = jnp.floor(
        lax.dot_general(inc, ones_col, (((0,), (0,)), ((), ())),
                        preferred_element_type=jnp.float32) + 0.5)
    nb = ((counts.astype(jnp.int32) + (BLK - 1)) >> 7).astype(jnp.float32)
    r64 = lax.broadcasted_iota(jnp.int32, (E, E), 0)
    c64 = lax.broadcasted_iota(jnp.int32, (E, E), 1)
    tri64 = (r64 >= c64).astype(jnp.float32)
    cnb = jnp.floor(
        lax.dot_general(tri64, nb, (((1,), (0,)), ((), ())),
                        preferred_element_type=jnp.float32) + 0.5)
    bs = cnb - nb
    tot = cnb[E - 1:E, 0:1]
    j96 = lax.broadcasted_iota(jnp.int32, (1, NBLK), 1).astype(jnp.float32)
    eobs = jnp.sum((cnb <= j96).astype(jnp.float32), axis=0, keepdims=True)
    e_col = lax.broadcasted_iota(jnp.int32, (E, 1), 0).astype(jnp.float32)
    last_e = jnp.max(jnp.where(counts > 0, e_col, -1.0), axis=0, keepdims=True)
    eob_ref[...] = jnp.where(j96 < tot, eobs, last_e).astype(jnp.int32)
    tot_ref[...] = jnp.broadcast_to(tot, (1, 16)).astype(jnp.int32)
    rows4 = lax.dot_general(
        jnp.concatenate([q1, q2, m1, m2], axis=1), eye,
        (((0,), (0,)), ((), ())),
        preferred_element_type=jnp.float32,
        precision=lax.Precision.HIGHEST)
    bs0 = lax.dot_general(bs, oh1, (((0,), (1,)), ((), ())),
                          preferred_element_type=jnp.float32,
                          precision=lax.Precision.HIGHEST)
    bs1 = lax.dot_general(bs, oh2, (((0,), (1,)), ((), ())),
                          preferred_element_type=jnp.float32,
                          precision=lax.Precision.HIGHEST)
    pp0 = (bs0 * BLK + rows4[0:1, :] + 0.5).astype(jnp.int32)
    pp1 = (bs1 * BLK + rows4[1:2, :] + 0.5).astype(jnp.int32)
    pos0_ref[...] = pp0.reshape(T)
    pos1_ref[...] = pp1.reshape(T)
    w0_ref[...] = rows4[2:3, :].reshape(T)
    w1_ref[...] = rows4[3:4, :].reshape(T)
    pp = jnp.concatenate([pp0, pp1], axis=1)
    g = pp >> 7
    rr_ref[...] = (((g & 31) << 3) + (g >> 5)).reshape(2 * T)
    cc_ref[...] = (pp & (BLK - 1)).reshape(2 * T)


def _gate(z, Wg, bg):
    return pl.pallas_call(
        _gate_body,
        grid=(1,),
        in_specs=[
            pl.BlockSpec((T, D), lambda i: (0, 0)),
            pl.BlockSpec((E, D), lambda i: (0, 0)),
            pl.BlockSpec((1, E), lambda i: (0, 0)),
        ],
        out_specs=[
            pl.BlockSpec((T,), lambda i: (0,)),
            pl.BlockSpec((T,), lambda i: (0,)),
            pl.BlockSpec((T,), lambda i: (0,)),
            pl.BlockSpec((T,), lambda i: (0,)),
            pl.BlockSpec((2 * T,), lambda i: (0,)),
            pl.BlockSpec((2 * T,), lambda i: (0,)),
            pl.BlockSpec((1, NBLK), lambda i: (0, 0)),
            pl.BlockSpec((1, 16), lambda i: (0, 0)),
        ],
        out_shape=[
            jax.ShapeDtypeStruct((T,), jnp.int32),
            jax.ShapeDtypeStruct((T,), jnp.int32),
            jax.ShapeDtypeStruct((T,), jnp.float32),
            jax.ShapeDtypeStruct((T,), jnp.float32),
            jax.ShapeDtypeStruct((2 * T,), jnp.int32),
            jax.ShapeDtypeStruct((2 * T,), jnp.int32),
            jax.ShapeDtypeStruct((1, NBLK), jnp.int32),
            jax.ShapeDtypeStruct((1, 16), jnp.int32),
        ],
    )(z, Wg, bg.reshape(1, E))


def _sc_gather(z, gimg, tot):
    mesh = plsc.VectorSubcoreMesh(core_axis_name="c", subcore_axis_name="s")
    n = NBLK * BLK
    nw = 32
    nch = n // nw // GW

    @functools.partial(
        pl.kernel,
        out_type=jax.ShapeDtypeStruct((n, D), jnp.float32),
        mesh=mesh,
        compiler_params=_sc_params(),
        scratch_types=[
            pltpu.VMEM((8, GW), jnp.int32),
            pltpu.VMEM((GW, D), jnp.float32),
            pltpu.VMEM((1, 16), jnp.int32),
            pltpu.SemaphoreType.DMA,
        ],
    )
    def k(z_hbm, g_hbm, t_hbm, x_hbm, idxs, buf, bnd, sem):
        wid = lax.axis_index("s") * 2 + lax.axis_index("c")
        pltpu.sync_copy(t_hbm, bnd)
        pltpu.sync_copy(g_hbm.at[wid], idxs)
        nblocks = bnd[0, :][0]
        for c in range(nch):
            @pl.when(c * nw + wid < nblocks)
            def _():
                pltpu.async_copy(z_hbm.at[idxs.at[c]], buf, sem).wait()
                pltpu.sync_copy(
                    buf, x_hbm.at[pl.ds((c * nw + wid) * GW, GW)])

    return k(z, gimg, tot)


def _ffn_body(eob_ref, tot_ref, x_ref, w1_ref, b1_ref, w2_ref, b2_ref, y_ref):
    j = pl.program_id(0)

    @pl.when(j < tot_ref[0, 0])
    def _():
        x = x_ref[...]
        h = jnp.maximum(
            lax.dot_general(x, w1_ref[0], (((1,), (0,)), ((), ())),
                            preferred_element_type=jnp.float32)
            + b1_ref[0], 0.0)
        y_ref[...] = lax.dot_general(
            h, w2_ref[0], (((1,), (0,)), ((), ())),
            preferred_element_type=jnp.float32) + b2_ref[0]


def _ffn(X, W1, b1, W2, b2, eob, tot):
    grid_spec = pltpu.PrefetchScalarGridSpec(
        num_scalar_prefetch=2,
        grid=(NBLK,),
        in_specs=[
            pl.BlockSpec((BLK, D), lambda j, eob, tot: (j, 0)),
            pl.BlockSpec((1, D, H), lambda j, eob, tot: (eob[0, j], 0, 0)),
            pl.BlockSpec((1, 1, H), lambda j, eob, tot: (eob[0, j], 0, 0)),
            pl.BlockSpec((1, H, D), lambda j, eob, tot: (eob[0, j], 0, 0)),
            pl.BlockSpec((1, 1, D), lambda j, eob, tot: (eob[0, j], 0, 0)),
        ],
        out_specs=pl.BlockSpec((BLK, D), lambda j, eob, tot: (j, 0)),
    )
    return pl.pallas_call(
        _ffn_body,
        grid_spec=grid_spec,
        out_shape=jax.ShapeDtypeStruct((NBLK * BLK, D), jnp.float32),
    )(eob, tot, X, W1, b1.reshape(E, 1, H), W2, b2.reshape(E, 1, D))


def _sc_combine(Y, p0, p1, w0, w1):
    mesh = plsc.VectorSubcoreMesh(core_axis_name="c", subcore_axis_name="s")
    nw = 32
    per = T // nw

    @functools.partial(
        pl.kernel,
        out_type=jax.ShapeDtypeStruct((T, D), jnp.float32),
        mesh=mesh,
        compiler_params=_sc_params(),
        scratch_types=[
            pltpu.VMEM((per,), jnp.int32),
            pltpu.VMEM((per,), jnp.int32),
            pltpu.VMEM((per,), jnp.float32),
            pltpu.VMEM((per,), jnp.float32),
            pltpu.VMEM((per, D), jnp.float32),
            pltpu.VMEM((per, D), jnp.float32),
            pltpu.SemaphoreType.DMA,
        ],
    )
    def k(y_hbm, p0_hbm, p1_hbm, w0_hbm, w1_hbm, o_hbm,
          i0, i1, iw0, iw1, b0, b1, sem):
        wid = lax.axis_index("s") * 2 + lax.axis_index("c")
        base = wid * per
        pltpu.sync_copy(p0_hbm.at[pl.ds(base, per)], i0)
        pltpu.sync_copy(p1_hbm.at[pl.ds(base, per)], i1)
        pltpu.sync_copy(w0_hbm.at[pl.ds(base, per)], iw0)
        pltpu.sync_copy(w1_hbm.at[pl.ds(base, per)], iw1)
        pltpu.async_copy(y_hbm.at[i0], b0, sem).wait()
        pltpu.async_copy(y_hbm.at[i1], b1, sem).wait()

        @pl.loop(0, per)
        def _(i):
            bidx = jnp.full((16,), i, jnp.int32)
            wv0 = plsc.load_gather(iw0, [bidx])
            wv1 = plsc.load_gather(iw1, [bidx])

            @pl.loop(0, D, step=16)
            def _(c):
                b0[i, pl.ds(c, 16)] = (wv0 * b0[i, pl.ds(c, 16)]
                                       + wv1 * b1[i, pl.ds(c, 16)])

        pltpu.sync_copy(b0, o_hbm.at[pl.ds(base, per)])

    return k(Y, p0, p1, w0, w1)


def kernel(z, Wg, bg, W1, b1, W2, b2):
    pos0, pos1, w0, w1, rr, cc, eob, tot = _gate(z, Wg, bg)

    base = (jnp.arange(256 * GW, dtype=jnp.int32) % T).reshape(256, GW)
    vals = jnp.tile(jnp.arange(T, dtype=jnp.int32), K)
    gimg = base.at[rr, cc].set(vals).reshape(32, 8, GW)

    X = _sc_gather(z, gimg, tot)
    Y = _ffn(X, W1, b1, W2, b2, eob, tot)
    return _sc_combine(Y, pos0, pos1, w0, w1)


# speedup vs baseline: 5.3066x; 1.2995x over previous
import dataclasses
import functools

import jax
import jax.numpy as jnp
from jax import lax
from jax.experimental import pallas as pl
from jax.experimental.pallas import tpu as pltpu
from jax.experimental.pallas import tpu_sc as plsc

E = 64
K = 2
D = 768
H = 768
T = 2048
BLK = 128
NBLK = 96
GW = 128


def _sc_params():
    cp = pltpu.CompilerParams()
    if "needs_layout_passes" in pltpu.CompilerParams.__dataclass_fields__:
        cp = dataclasses.replace(cp, needs_layout_passes=False)
    return cp


def _gate_body(z_ref, wg_ref, bg_ref, pos0_ref, pos1_ref, w0_ref, w1_ref,
               eob_ref, tot_ref):
    z = z_ref[...]
    logits = lax.dot_general(z, wg_ref[...], (((1,), (1,)), ((), ())),
                             preferred_element_type=jnp.float32)
    logits = logits + bg_ref[...]
    m = jnp.max(logits, axis=1, keepdims=True)
    ex = jnp.exp(logits - m)
    w = ex / jnp.sum(ex, axis=1, keepdims=True)
    iota = lax.broadcasted_iota(jnp.int32, w.shape, 1)
    m1 = jnp.max(w, axis=1, keepdims=True)
    a1 = jnp.min(jnp.where(w == m1, iota, E), axis=1, keepdims=True)
    wm = jnp.where(iota == a1, -jnp.inf, w)
    m2 = jnp.max(wm, axis=1, keepdims=True)
    a2 = jnp.min(jnp.where(wm == m2, iota, E), axis=1, keepdims=True)
    oh1 = (iota == a1).astype(jnp.float32)
    oh2 = (iota == a2).astype(jnp.float32)
    inc = oh1 + oh2
    r = lax.broadcasted_iota(jnp.int32, (T, T), 0)
    c = lax.broadcasted_iota(jnp.int32, (T, T), 1)
    tri = (r > c).astype(jnp.float32)
    eye = (r == c).astype(jnp.float32)
    excl = lax.dot_general(tri, inc, (((1,), (0,)), ((), ())),
                           preferred_element_type=jnp.float32)
    q1 = jnp.floor(jnp.sum(excl * oh1, axis=1, keepdims=True) + 0.5)
    q2 = jnp.floor(jnp.sum(excl * oh2, axis=1, keepdims=True) + 0.5)
    ones_col = jnp.ones((T, 1), jnp.float32)
    counts = jnp.floor(
        lax.dot_general(inc, ones_col, (((0,), (0,)), ((), ())),
                        preferred_element_type=jnp.float32) + 0.5)
    nb = ((counts.astype(jnp.int32) + (BLK - 1)) >> 7).astype(jnp.float32)
    r64 = lax.broadcasted_iota(jnp.int32, (E, E), 0)
    c64 = lax.broadcasted_iota(jnp.int32, (E, E), 1)
    tri64 = (r64 >= c64).astype(jnp.float32)
    cnb = jnp.floor(
        lax.dot_general(tri64, nb, (((1,), (0,)), ((), ())),
                        preferred_element_type=jnp.float32) + 0.5)
    bs = cnb - nb
    tot = cnb[E - 1:E, 0:1]
    j96 = lax.broadcasted_iota(jnp.int32, (1, NBLK), 1).astype(jnp.float32)
    eobs = jnp.sum((cnb <= j96).astype(jnp.float32), axis=0, keepdims=True)
    e_col = lax.broadcasted_iota(jnp.int32, (E, 1), 0).astype(jnp.float32)
    last_e = jnp.max(jnp.where(counts > 0, e_col, -1.0), axis=0, keepdims=True)
    eob_ref[...] = jnp.where(j96 < tot, eobs, last_e).astype(jnp.int32)
    tot_ref[...] = jnp.broadcast_to(tot, (1, 16)).astype(jnp.int32)
    rows4 = lax.dot_general(
        jnp.concatenate([q1, q2, m1, m2], axis=1), eye,
        (((0,), (0,)), ((), ())),
        preferred_element_type=jnp.float32,
        precision=lax.Precision.HIGHEST)
    bs0 = lax.dot_general(bs, oh1, (((0,), (1,)), ((), ())),
                          preferred_element_type=jnp.float32,
                          precision=lax.Precision.HIGHEST)
    bs1 = lax.dot_general(bs, oh2, (((0,), (1,)), ((), ())),
                          preferred_element_type=jnp.float32,
                          precision=lax.Precision.HIGHEST)
    pp0 = (bs0 * BLK + rows4[0:1, :] + 0.5).astype(jnp.int32)
    pp1 = (bs1 * BLK + rows4[1:2, :] + 0.5).astype(jnp.int32)
    pos0_ref[...] = pp0.reshape(T)
    pos1_ref[...] = pp1.reshape(T)
    w0_ref[...] = rows4[2:3, :].reshape(T)
    w1_ref[...] = rows4[3:4, :].reshape(T)


def _gate(z, Wg, bg):
    return pl.pallas_call(
        _gate_body,
        grid=(1,),
        in_specs=[
            pl.BlockSpec((T, D), lambda i: (0, 0)),
            pl.BlockSpec((E, D), lambda i: (0, 0)),
            pl.BlockSpec((1, E), lambda i: (0, 0)),
        ],
        out_specs=[
            pl.BlockSpec((T,), lambda i: (0,)),
            pl.BlockSpec((T,), lambda i: (0,)),
            pl.BlockSpec((T,), lambda i: (0,)),
            pl.BlockSpec((T,), lambda i: (0,)),
            pl.BlockSpec((1, NBLK), lambda i: (0, 0)),
            pl.BlockSpec((1, 16), lambda i: (0, 0)),
        ],
        out_shape=[
            jax.ShapeDtypeStruct((T,), jnp.int32),
            jax.ShapeDtypeStruct((T,), jnp.int32),
            jax.ShapeDtypeStruct((T,), jnp.float32),
            jax.ShapeDtypeStruct((T,), jnp.float32),
            jax.ShapeDtypeStruct((1, NBLK), jnp.int32),
            jax.ShapeDtypeStruct((1, 16), jnp.int32),
        ],
    )(z, Wg, bg.reshape(1, E))


def _sc_dispatch(z, p0, p1):
    mesh = plsc.VectorSubcoreMesh(core_axis_name="c", subcore_axis_name="s")
    nw = 32
    per = T // nw

    @functools.partial(
        pl.kernel,
        out_type=jax.ShapeDtypeStruct((NBLK * BLK, D), jnp.float32),
        mesh=mesh,
        compiler_params=_sc_params(),
        scratch_types=[
            pltpu.VMEM((per,), jnp.int32),
            pltpu.VMEM((per,), jnp.int32),
            pltpu.VMEM((per, D), jnp.float32),
        ],
    )
    def k(z_hbm, p0_hbm, p1_hbm, x_hbm, i0, i1, buf):
        wid = lax.axis_index("s") * 2 + lax.axis_index("c")
        base = wid * per
        pltpu.sync_copy(p0_hbm.at[pl.ds(base, per)], i0)
        pltpu.sync_copy(p1_hbm.at[pl.ds(base, per)], i1)
        pltpu.sync_copy(z_hbm.at[pl.ds(base, per)], buf)
        pltpu.sync_copy(buf, x_hbm.at[i0])
        pltpu.sync_copy(buf, x_hbm.at[i1])

    return k(z, p0, p1)


def _ffn_body(eob_ref, tot_ref, x_ref, w1_ref, b1_ref, w2_ref, b2_ref, y_ref):
    j = pl.program_id(0)

    @pl.when(j < tot_ref[0, 0])
    def _():
        x = x_ref[...]
        h = jnp.maximum(
            lax.dot_general(x, w1_ref[0], (((1,), (0,)), ((), ())),
                            preferred_element_type=jnp.float32)
            + b1_ref[0], 0.0)
        y_ref[...] = lax.dot_general(
            h, w2_ref[0], (((1,), (0,)), ((), ())),
            preferred_element_type=jnp.float32) + b2_ref[0]


def _ffn(X, W1, b1, W2, b2, eob, tot):
    grid_spec = pltpu.PrefetchScalarGridSpec(
        num_scalar_prefetch=2,
        grid=(NBLK,),
        in_specs=[
            pl.BlockSpec((BLK, D),
                         lambda j, eob, tot: (lax.select(j < tot[0, 0], j, 0),
                                              0)),
            pl.BlockSpec((1, D, H), lambda j, eob, tot: (eob[0, j], 0, 0)),
            pl.BlockSpec((1, 1, H), lambda j, eob, tot: (eob[0, j], 0, 0)),
            pl.BlockSpec((1, H, D), lambda j, eob, tot: (eob[0, j], 0, 0)),
            pl.BlockSpec((1, 1, D), lambda j, eob, tot: (eob[0, j], 0, 0)),
        ],
        out_specs=pl.BlockSpec(
            (BLK, D),
            lambda j, eob, tot: (lax.select(j < tot[0, 0], j, NBLK), 0)),
    )
    return pl.pallas_call(
        _ffn_body,
        grid_spec=grid_spec,
        out_shape=jax.ShapeDtypeStruct(((NBLK + 1) * BLK, D), jnp.float32),
    )(eob, tot, X, W1, b1.reshape(E, 1, H), W2, b2.reshape(E, 1, D))


def _sc_combine(Y, p0, p1, w0, w1):
    mesh = plsc.VectorSubcoreMesh(core_axis_name="c", subcore_axis_name="s")
    nw = 32
    per = T // nw

    @functools.partial(
        pl.kernel,
        out_type=jax.ShapeDtypeStruct((T, D), jnp.float32),
        mesh=mesh,
        compiler_params=_sc_params(),
        scratch_types=[
            pltpu.VMEM((per,), jnp.int32),
            pltpu.VMEM((per,), jnp.int32),
            pltpu.VMEM((per,), jnp.float32),
            pltpu.VMEM((per,), jnp.float32),
            pltpu.VMEM((per, D), jnp.float32),
            pltpu.VMEM((per, D), jnp.float32),
            pltpu.SemaphoreType.DMA,
        ],
    )
    def k(y_hbm, p0_hbm, p1_hbm, w0_hbm, w1_hbm, o_hbm,
          i0, i1, iw0, iw1, b0, b1, sem):
        wid = lax.axis_index("s") * 2 + lax.axis_index("c")
        base = wid * per
        pltpu.sync_copy(p0_hbm.at[pl.ds(base, per)], i0)
        pltpu.sync_copy(p1_hbm.at[pl.ds(base, per)], i1)
        pltpu.sync_copy(w0_hbm.at[pl.ds(base, per)], iw0)
        pltpu.sync_copy(w1_hbm.at[pl.ds(base, per)], iw1)
        pltpu.async_copy(y_hbm.at[i0], b0, sem).wait()
        pltpu.async_copy(y_hbm.at[i1], b1, sem).wait()

        @pl.loop(0, per)
        def _(i):
            bidx = jnp.full((16,), i, jnp.int32)
            wv0 = plsc.load_gather(iw0, [bidx])
            wv1 = plsc.load_gather(iw1, [bidx])

            @pl.loop(0, D, step=16)
            def _(c):
                b0[i, pl.ds(c, 16)] = (wv0 * b0[i, pl.ds(c, 16)]
                                       + wv1 * b1[i, pl.ds(c, 16)])

        pltpu.sync_copy(b0, o_hbm.at[pl.ds(base, per)])

    return k(Y, p0, p1, w0, w1)


def kernel(z, Wg, bg, W1, b1, W2, b2):
    pos0, pos1, w0, w1, eob, tot = _gate(z, Wg, bg)
    X = _sc_dispatch(z, pos0, pos1)
    Y = _ffn(X, W1, b1, W2, b2, eob, tot)
    return _sc_combine(Y, pos0, pos1, w0, w1)


# speedup vs baseline: 5.7857x; 1.0903x over previous
import dataclasses
import functools

import jax
import jax.numpy as jnp
from jax import lax
from jax.experimental import pallas as pl
from jax.experimental.pallas import tpu as pltpu
from jax.experimental.pallas import tpu_sc as plsc

E = 64
K = 2
D = 768
H = 768
T = 2048
BLK = 128
NBLK = 96
GW = 128


def _sc_params():
    cp = pltpu.CompilerParams()
    if "needs_layout_passes" in pltpu.CompilerParams.__dataclass_fields__:
        cp = dataclasses.replace(cp, needs_layout_passes=False)
    return cp


def _gate_body(z_ref, wg_ref, bg_ref, pos0_ref, pos1_ref, w0_ref, w1_ref,
               eob_ref, tot_ref):
    z = z_ref[...]
    logits = lax.dot_general(z, wg_ref[...], (((1,), (1,)), ((), ())),
                             preferred_element_type=jnp.float32)
    logits = logits + bg_ref[...]
    m = jnp.max(logits, axis=1, keepdims=True)
    ex = jnp.exp(logits - m)
    w = ex / jnp.sum(ex, axis=1, keepdims=True)
    iota = lax.broadcasted_iota(jnp.int32, w.shape, 1)
    m1 = jnp.max(w, axis=1, keepdims=True)
    a1 = jnp.min(jnp.where(w == m1, iota, E), axis=1, keepdims=True)
    wm = jnp.where(iota == a1, -jnp.inf, w)
    m2 = jnp.max(wm, axis=1, keepdims=True)
    a2 = jnp.min(jnp.where(wm == m2, iota, E), axis=1, keepdims=True)
    oh1 = (iota == a1).astype(jnp.float32)
    oh2 = (iota == a2).astype(jnp.float32)
    inc = oh1 + oh2
    r = lax.broadcasted_iota(jnp.int32, (T, T), 0)
    c = lax.broadcasted_iota(jnp.int32, (T, T), 1)
    tri = (r > c).astype(jnp.float32)
    eye = (r == c).astype(jnp.float32)
    excl = lax.dot_general(tri, inc, (((1,), (0,)), ((), ())),
                           preferred_element_type=jnp.float32)
    q1 = jnp.floor(jnp.sum(excl * oh1, axis=1, keepdims=True) + 0.5)
    q2 = jnp.floor(jnp.sum(excl * oh2, axis=1, keepdims=True) + 0.5)
    ones_col = jnp.ones((T, 1), jnp.float32)
    counts = jnp.floor(
        lax.dot_general(inc, ones_col, (((0,), (0,)), ((), ())),
                        preferred_element_type=jnp.float32) + 0.5)
    nb = ((counts.astype(jnp.int32) + (BLK - 1)) >> 7).astype(jnp.float32)
    r64 = lax.broadcasted_iota(jnp.int32, (E, E), 0)
    c64 = lax.broadcasted_iota(jnp.int32, (E, E), 1)
    tri64 = (r64 >= c64).astype(jnp.float32)
    cnb = jnp.floor(
        lax.dot_general(tri64, nb, (((1,), (0,)), ((), ())),
                        preferred_element_type=jnp.float32) + 0.5)
    bs = cnb - nb
    tot = cnb[E - 1:E, 0:1]
    j96 = lax.broadcasted_iota(jnp.int32, (1, NBLK), 1).astype(jnp.float32)
    eobs = jnp.sum((cnb <= j96).astype(jnp.float32), axis=0, keepdims=True)
    e_col = lax.broadcasted_iota(jnp.int32, (E, 1), 0).astype(jnp.float32)
    last_e = jnp.max(jnp.where(counts > 0, e_col, -1.0), axis=0, keepdims=True)
    eob_ref[...] = jnp.where(j96 < tot, eobs, last_e).astype(jnp.int32)
    tot_ref[...] = jnp.broadcast_to(tot, (1, 16)).astype(jnp.int32)
    rows4 = lax.dot_general(
        jnp.concatenate([q1, q2, m1, m2], axis=1), eye,
        (((0,), (0,)), ((), ())),
        preferred_element_type=jnp.float32,
        precision=lax.Precision.HIGHEST)
    bs0 = lax.dot_general(bs, oh1, (((0,), (1,)), ((), ())),
                          preferred_element_type=jnp.float32,
                          precision=lax.Precision.HIGHEST)
    bs1 = lax.dot_general(bs, oh2, (((0,), (1,)), ((), ())),
                          preferred_element_type=jnp.float32,
                          precision=lax.Precision.HIGHEST)
    pp0 = (bs0 * BLK + rows4[0:1, :] + 0.5).astype(jnp.int32)
    pp1 = (bs1 * BLK + rows4[1:2, :] + 0.5).astype(jnp.int32)
    pos0_ref[...] = pp0.reshape(T)
    pos1_ref[...] = pp1.reshape(T)
    w0_ref[...] = rows4[2:3, :].reshape(T)
    w1_ref[...] = rows4[3:4, :].reshape(T)


def _gate(z, Wg, bg):
    return pl.pallas_call(
        _gate_body,
        grid=(1,),
        in_specs=[
            pl.BlockSpec((T, D), lambda i: (0, 0)),
            pl.BlockSpec((E, D), lambda i: (0, 0)),
            pl.BlockSpec((1, E), lambda i: (0, 0)),
        ],
        out_specs=[
            pl.BlockSpec((T,), lambda i: (0,)),
            pl.BlockSpec((T,), lambda i: (0,)),
            pl.BlockSpec((T,), lambda i: (0,)),
            pl.BlockSpec((T,), lambda i: (0,)),
            pl.BlockSpec((1, NBLK), lambda i: (0, 0)),
            pl.BlockSpec((1, 16), lambda i: (0, 0)),
        ],
        out_shape=[
            jax.ShapeDtypeStruct((T,), jnp.int32),
            jax.ShapeDtypeStruct((T,), jnp.int32),
            jax.ShapeDtypeStruct((T,), jnp.float32),
            jax.ShapeDtypeStruct((T,), jnp.float32),
            jax.ShapeDtypeStruct((1, NBLK), jnp.int32),
            jax.ShapeDtypeStruct((1, 16), jnp.int32),
        ],
    )(z, Wg, bg.reshape(1, E))


def _sc_dispatch(z, p0, p1):
    mesh = plsc.VectorSubcoreMesh(core_axis_name="c", subcore_axis_name="s")
    nw = 32
    per = T // nw

    @functools.partial(
        pl.kernel,
        out_type=jax.ShapeDtypeStruct((NBLK * BLK, D), jnp.float32),
        mesh=mesh,
        compiler_params=_sc_params(),
        scratch_types=[
            pltpu.VMEM((per,), jnp.int32),
            pltpu.VMEM((per,), jnp.int32),
            pltpu.VMEM((per, D), jnp.float32),
            pltpu.SemaphoreType.DMA,
            pltpu.SemaphoreType.DMA,
            pltpu.SemaphoreType.DMA,
        ],
    )
    def k(z_hbm, p0_hbm, p1_hbm, x_hbm, i0, i1, buf, s0, s1, s2):
        wid = lax.axis_index("s") * 2 + lax.axis_index("c")
        base = wid * per
        c0 = pltpu.async_copy(p0_hbm.at[pl.ds(base, per)], i0, s0)
        c1 = pltpu.async_copy(p1_hbm.at[pl.ds(base, per)], i1, s1)
        c2 = pltpu.async_copy(z_hbm.at[pl.ds(base, per)], buf, s2)
        c0.wait()
        c2.wait()
        c3 = pltpu.async_copy(buf, x_hbm.at[i0], s0)
        c1.wait()
        c4 = pltpu.async_copy(buf, x_hbm.at[i1], s1)
        c3.wait()
        c4.wait()

    return k(z, p0, p1)


def _ffn_body(eob_ref, tot_ref, x_ref, w1_ref, b1_ref, w2_ref, b2_ref, y_ref):
    j = pl.program_id(0)

    @pl.when(j < tot_ref[0, 0])
    def _():
        x = x_ref[...]
        h = jnp.maximum(
            lax.dot_general(x, w1_ref[0], (((1,), (0,)), ((), ())),
                            preferred_element_type=jnp.float32)
            + b1_ref[0], 0.0)
        y_ref[...] = lax.dot_general(
            h, w2_ref[0], (((1,), (0,)), ((), ())),
            preferred_element_type=jnp.float32) + b2_ref[0]


def _ffn(X, W1, b1, W2, b2, eob, tot):
    grid_spec = pltpu.PrefetchScalarGridSpec(
        num_scalar_prefetch=2,
        grid=(NBLK,),
        in_specs=[
            pl.BlockSpec((BLK, D),
                         lambda j, eob, tot: (lax.select(j < tot[0, 0], j, 0),
                                              0)),
            pl.BlockSpec((1, D, H), lambda j, eob, tot: (eob[0, j], 0, 0)),
            pl.BlockSpec((1, 1, H), lambda j, eob, tot: (eob[0, j], 0, 0)),
            pl.BlockSpec((1, H, D), lambda j, eob, tot: (eob[0, j], 0, 0)),
            pl.BlockSpec((1, 1, D), lambda j, eob, tot: (eob[0, j], 0, 0)),
        ],
        out_specs=pl.BlockSpec(
            (BLK, D),
            lambda j, eob, tot: (lax.select(j < tot[0, 0], j, NBLK), 0)),
    )
    return pl.pallas_call(
        _ffn_body,
        grid_spec=grid_spec,
        out_shape=jax.ShapeDtypeStruct(((NBLK + 1) * BLK, D), jnp.float32),
    )(eob, tot, X, W1, b1.reshape(E, 1, H), W2, b2.reshape(E, 1, D))


def _sc_combine(Y, p0, p1, w0, w1):
    mesh = plsc.VectorSubcoreMesh(core_axis_name="c", subcore_axis_name="s")
    nw = 32
    per = T // nw

    @functools.partial(
        pl.kernel,
        out_type=jax.ShapeDtypeStruct((T, D), jnp.float32),
        mesh=mesh,
        compiler_params=_sc_params(),
        scratch_types=[
            pltpu.VMEM((per,), jnp.int32),
            pltpu.VMEM((per,), jnp.int32),
            pltpu.VMEM((per,), jnp.float32),
            pltpu.VMEM((per,), jnp.float32),
            pltpu.VMEM((per, D), jnp.float32),
            pltpu.VMEM((per, D), jnp.float32),
            pltpu.SemaphoreType.DMA,
            pltpu.SemaphoreType.DMA,
        ],
    )
    def k(y_hbm, p0_hbm, p1_hbm, w0_hbm, w1_hbm, o_hbm,
          i0, i1, iw0, iw1, b0, b1, s0, s1):
        wid = lax.axis_index("s") * 2 + lax.axis_index("c")
        base = wid * per
        c0 = pltpu.async_copy(p0_hbm.at[pl.ds(base, per)], i0, s0)
        c1 = pltpu.async_copy(p1_hbm.at[pl.ds(base, per)], i1, s1)
        pltpu.sync_copy(w0_hbm.at[pl.ds(base, per)], iw0)
        pltpu.sync_copy(w1_hbm.at[pl.ds(base, per)], iw1)
        c0.wait()
        g0 = pltpu.async_copy(y_hbm.at[i0], b0, s0)
        c1.wait()
        g1 = pltpu.async_copy(y_hbm.at[i1], b1, s1)
        g0.wait()
        g1.wait()

        @pl.loop(0, per)
        def _(i):
            bidx = jnp.full((16,), i, jnp.int32)
            wv0 = plsc.load_gather(iw0, [bidx])
            wv1 = plsc.load_gather(iw1, [bidx])
            for c in range(0, D, 16):
                b0[i, pl.ds(c, 16)] = (wv0 * b0[i, pl.ds(c, 16)]
                                       + wv1 * b1[i, pl.ds(c, 16)])

        pltpu.sync_copy(b0, o_hbm.at[pl.ds(base, per)])

    return k(Y, p0, p1, w0, w1)


def kernel(z, Wg, bg, W1, b1, W2, b2):
    pos0, pos1, w0, w1, eob, tot = _gate(z, Wg, bg)
    X = _sc_dispatch(z, pos0, pos1)
    Y = _ffn(X, W1, b1, W2, b2, eob, tot)
    return _sc_combine(Y, pos0, pos1, w0, w1)
